# baseline (speedup 1.0000x reference)
import jax
import jax.numpy as jnp
from jax import lax
from jax.experimental import pallas as pl
from jax.experimental.pallas import tpu as pltpu
from jax.experimental.pallas import tpu_sc as plsc

N = 10000
F = 128
H = 8
O = 8
D1 = H * O
D2 = 128
SEQ = 4
NEG = 0.2
NROWS = 10112
C = 128
NT = 32
NCH = 82
PT = NCH * C
EPAD = NT * PT
RPT = NROWS // 16
W1R = D1 + 16
W2R = D2 + 16
HALF = NROWS // 2
R2 = 5088
QTR = NROWS // 4
QR2 = 2544

f32 = jnp.float32
i32 = jnp.int32

_mesh = plsc.VectorSubcoreMesh(core_axis_name="c", subcore_axis_name="s")
_SC_PARAMS = pltpu.CompilerParams(use_tc_tiling_on_sc=False)
_VMEM_BIG = pltpu.CompilerParams(vmem_limit_bytes=63 * 1024 * 1024)

_GATHER_DN = lax.GatherDimensionNumbers(
    offset_dims=(), collapsed_slice_dims=(0,), start_index_map=(0,))


def _lane_gather(v, idx):
    return lax.gather(v, idx[:, None], _GATHER_DN, slice_sizes=(1,),
                      mode=lax.GatherScatterMode.PROMISE_IN_BOUNDS)


def _make_sc_body(DH, WR, NPH, PHR, DUMMY, DRPT, ZR, ZC, heads):
    def body(idxT, aspf, adpf, hf, out,
             idxb, dscat, asr, adr, hr, sbuf, zbuf, acc,
             semi, semg, sems):
        cid = lax.axis_index("c")
        sid = lax.axis_index("s")
        wid = sid * 2 + cid
        zv = jnp.zeros((16,), f32)

        @pl.loop(0, ZR)
        def _(r):
            @pl.loop(0, WR, step=16)
            def _(c0):
                zbuf[r, pl.ds(c0, 16)] = zv

        def issue_idx(par, t, g):
            ci = (t * NT + wid) * NCH + g
            pltpu.async_copy(idxT.at[ci], idxb.at[par], semi.at[par])

        def wait_idx(par):
            pltpu.make_async_copy(idxT.at[0], idxb.at[par], semi.at[par]).wait()

        def issue_gathers(par):
            pltpu.async_copy(aspf.at[idxb.at[par, 0]], asr.at[par], semg.at[par])
            pltpu.async_copy(adpf.at[idxb.at[par, 1]], adr.at[par], semg.at[par])
            pltpu.async_copy(hf.at[idxb.at[par, 0]], hr.at[par], semg.at[par])

        def wait_gathers(par):
            pltpu.make_async_copy(aspf.at[idxb.at[par, 0]], asr.at[par],
                                  semg.at[par]).wait()
            pltpu.make_async_copy(adpf.at[idxb.at[par, 1]], adr.at[par],
                                  semg.at[par]).wait()
            pltpu.make_async_copy(hf.at[idxb.at[par, 0]], hr.at[par],
                                  semg.at[par]).wait()

        def issue_scatter(par):
            pltpu.async_copy(sbuf.at[par], acc.at[dscat.at[par]], sems.at[par],
                             add=True)

        def wait_scatter(par):
            pltpu.make_async_copy(sbuf.at[par], acc.at[dscat.at[par]],
                                  sems.at[par]).wait()

        def compute(par, t, p):
            off = t * NROWS + p * PHR

            @pl.loop(0, C, step=16)
            def _(k):
                rel = idxb[par, 1, pl.ds(k, 16)] - off
                inh = (rel >= 0) & (rel < PHR)
                dscat[par, pl.ds(k, 16)] = jnp.where(inh, rel, DUMMY)

            pair = lax.iota(i32, 16) >> 3

            @pl.loop(0, C)
            def _(j):
                gv = asr[par, j] + adr[par, j]
                e = jnp.exp(jnp.where(gv > 0, gv, gv * NEG))
                if heads:
                    sbuf[par, j, pl.ds(DH, 16)] = e
                    for r in range(DH // 16):
                        wb = _lane_gather(e, pair + 2 * r)
                        sbuf[par, j, pl.ds(16 * r, 16)] = (
                            hr[par, j, pl.ds(16 * r, 16)] * wb)
                else:
                    lane0 = lax.iota(i32, 16) == 0
                    sbuf[par, j, pl.ds(DH, 16)] = jnp.where(lane0, e, 0.0)
                    for r in range(DH // 16):
                        sbuf[par, j, pl.ds(16 * r, 16)] = (
                            hr[par, j, pl.ds(16 * r, 16)] * e)

        @pl.loop(0, SEQ)
        def _(t):
            for p in range(NPH):
                for z in range(ZC):
                    pltpu.sync_copy(
                        zbuf, acc.at[pl.ds(sid * (ZR * ZC) + z * ZR, ZR)])
                plsc.subcore_barrier()

                issue_idx(0, t, 0)
                issue_idx(1, t, 1)
                wait_idx(0)
                issue_gathers(0)

                @pl.loop(0, NCH)
                def _(g):
                    par = g & 1
                    wait_gathers(par)
                    issue_idx(par, t, g + 2)

                    @pl.when(g + 1 < NCH)
                    def _():
                        wait_idx(1 - par)
                        issue_gathers(1 - par)

                    @pl.when(g >= 2)
                    def _():
                        wait_scatter(par)

                    compute(par, t, p)
                    issue_scatter(par)

                wait_scatter(0)
                wait_scatter(1)
                wait_idx(0)
                wait_idx(1)
                plsc.subcore_barrier()
                obase = (cid * SEQ + t) * NROWS + p * PHR + sid * DRPT
                pltpu.sync_copy(acc.at[pl.ds(sid * DRPT, DRPT)],
                                out.at[pl.ds(obase, DRPT)])
                plsc.subcore_barrier()

    return body


def _make_sc_kernel(DH, WR, NPH, PHR, DUMMY, DRPT, ACCR, ZR, ZC, heads):
    return pl.kernel(
        _make_sc_body(DH, WR, NPH, PHR, DUMMY, DRPT, ZR, ZC, heads),
        out_type=jax.ShapeDtypeStruct((2 * SEQ * NROWS, WR), f32),
        mesh=_mesh,
        scratch_types=[
            pltpu.VMEM((2, 2, C), i32), pltpu.VMEM((2, C), i32),
            pltpu.VMEM((2, C, 16), f32), pltpu.VMEM((2, C, 16), f32),
            pltpu.VMEM((2, C, DH), f32), pltpu.VMEM((2, C, WR), f32),
            pltpu.VMEM((ZR, WR), f32),
            pltpu.VMEM_SHARED((ACCR, WR), f32),
            pltpu.SemaphoreType.DMA((2,)), pltpu.SemaphoreType.DMA((2,)),
            pltpu.SemaphoreType.DMA((2,)),
        ],
        compiler_params=_SC_PARAMS,
    )


def _tc_embed1(x_ref, w_ref, h_ref):
    h = jnp.dot(x_ref[0], w_ref[...], preferred_element_type=f32)
    h_ref[...] = jnp.concatenate([h, jnp.zeros((NROWS - N, D1), f32)], 0)


def _tc_coef1(h_ref, ss_ref, sd_ref, as_ref, ad_ref):
    h = h_ref[...]
    a_s = jnp.dot(h, ss_ref[...], preferred_element_type=f32)
    a_d = jnp.dot(h, sd_ref[...], preferred_element_type=f32)
    z8 = jnp.zeros((NROWS, 8), f32)
    as_ref[...] = jnp.concatenate([a_s, z8], 1)
    ad_ref[...] = jnp.concatenate([a_d, z8], 1)


def _tc_embed2(p_ref, b1_ref, w2_ref, rx_ref, h2_ref):
    m = p_ref[0, 0] + p_ref[1, 0]
    msg = m[:, 0:D1]
    den = m[:, D1:D1 + H]
    den64 = jnp.dot(den, rx_ref[...], preferred_element_type=f32)
    y = msg / (den64 + 1e-16) + b1_ref[...]
    y = jnp.where(y > 0, y, jnp.exp(y) - 1.0)
    h2_ref[...] = jnp.dot(y, w2_ref[...],
                          preferred_element_type=f32)


def _tc_coef2(h_ref, as2_ref, ad2_ref, asb_ref, adb_ref):
    h2 = h_ref[...]
    a_s = jnp.sum(h2 * as2_ref[...], axis=-1, keepdims=True)
    a_d = jnp.sum(h2 * ad2_ref[...], axis=-1, keepdims=True)
    asb_ref[...] = jnp.broadcast_to(a_s, (NROWS, 16))
    adb_ref[...] = jnp.broadcast_to(a_d, (NROWS, 16))


def _tc_final(p_ref, b2_ref, o_ref):
    m = p_ref[0, 0] + p_ref[1, 0]
    v = m[0:N, 0:D2] / (m[0:N, D2:D2 + 1] + 1e-16) + b2_ref[...]
    mx = jnp.max(v, axis=-1, keepdims=True)
    s = v - mx
    o_ref[0] = s - jnp.log(jnp.sum(jnp.exp(s), axis=-1, keepdims=True))


def kernel(x, edge_index, W1, a_src1, a_dst1, b1, W2, a_src2, a_dst2, b2):
    loop_idx = jnp.arange(N, dtype=i32)
    ei = edge_index.astype(i32)
    npad = EPAD - (ei.shape[1] + N)
    padv = jnp.full((npad,), N, i32)
    src = jnp.concatenate([ei[0], loop_idx, padv])
    dst = jnp.concatenate([ei[1], loop_idx, padv])
    toff = (jnp.arange(SEQ, dtype=i32) * NROWS)[:, None]
    srcT = (src[None] + toff).reshape(SEQ, NT, NCH, C)
    dstT = (dst[None] + toff).reshape(SEQ, NT, NCH, C)
    idxT = jnp.stack([srcT, dstT], axis=3).reshape(SEQ * NT * NCH, 2, C)
    idxT = jnp.concatenate([idxT, jnp.full((2, 2, C), N, i32)], axis=0)

    sc_l1 = _make_sc_kernel(D1, W1R, 2, HALF, HALF, HALF // 16, R2,
                            159, 2, True)
    sc_l2 = _make_sc_kernel(D2, W2R, 4, QTR, QTR, QTR // 16, QR2,
                            159, 1, False)

    h1f = pl.pallas_call(
        _tc_embed1,
        grid=(SEQ,),
        in_specs=[
            pl.BlockSpec((1, N, F), lambda t: (t, 0, 0)),
            pl.BlockSpec((F, D1), lambda t: (0, 0)),
        ],
        out_specs=pl.BlockSpec((NROWS, D1), lambda t: (t, 0)),
        out_shape=jax.ShapeDtypeStruct((SEQ * NROWS, D1), f32),
    )(x, W1)

    eye = jnp.eye(H, dtype=f32)
    s_src = (a_src1[:, :, None] * eye[:, None, :]).reshape(D1, H)
    s_dst = (a_dst1[:, :, None] * eye[:, None, :]).reshape(D1, H)
    rx = jnp.repeat(eye, O, axis=1).reshape(H, D1)

    aspf, adpf = pl.pallas_call(
        _tc_coef1,
        grid=(SEQ,),
        in_specs=[
            pl.BlockSpec((NROWS, D1), lambda t: (t, 0)),
            pl.BlockSpec((D1, H), lambda t: (0, 0)),
            pl.BlockSpec((D1, H), lambda t: (0, 0)),
        ],
        out_specs=[
            pl.BlockSpec((NROWS, 16), lambda t: (t, 0)),
            pl.BlockSpec((NROWS, 16), lambda t: (t, 0)),
        ],
        out_shape=[
            jax.ShapeDtypeStruct((SEQ * NROWS, 16), f32),
            jax.ShapeDtypeStruct((SEQ * NROWS, 16), f32),
        ],
    )(h1f, s_src, s_dst)

    p1 = sc_l1(idxT, aspf, adpf, h1f)
    p1 = p1.reshape(2, SEQ, NROWS, W1R)

    h2f = pl.pallas_call(
        _tc_embed2,
        grid=(SEQ,),
        in_specs=[
            pl.BlockSpec((2, 1, NROWS, W1R), lambda t: (0, t, 0, 0)),
            pl.BlockSpec((1, D1), lambda t: (0, 0)),
            pl.BlockSpec((D1, D2), lambda t: (0, 0)),
            pl.BlockSpec((H, D1), lambda t: (0, 0)),
        ],
        out_specs=pl.BlockSpec((NROWS, D2), lambda t: (t, 0)),
        out_shape=jax.ShapeDtypeStruct((SEQ * NROWS, D2), f32),
        compiler_params=_VMEM_BIG,
    )(p1, b1.reshape(1, D1), W2, rx)

    asbf, adbf = pl.pallas_call(
        _tc_coef2,
        grid=(SEQ,),
        in_specs=[
            pl.BlockSpec((NROWS, D2), lambda t: (t, 0)),
            pl.BlockSpec((1, D2), lambda t: (0, 0)),
            pl.BlockSpec((1, D2), lambda t: (0, 0)),
        ],
        out_specs=[
            pl.BlockSpec((NROWS, 16), lambda t: (t, 0)),
            pl.BlockSpec((NROWS, 16), lambda t: (t, 0)),
        ],
        out_shape=[
            jax.ShapeDtypeStruct((SEQ * NROWS, 16), f32),
            jax.ShapeDtypeStruct((SEQ * NROWS, 16), f32),
        ],
    )(h2f, a_src2, a_dst2)

    p2 = sc_l2(idxT, asbf, adbf, h2f)
    p2 = p2.reshape(2, SEQ, NROWS, W2R)

    out = pl.pallas_call(
        _tc_final,
        grid=(SEQ,),
        in_specs=[
            pl.BlockSpec((2, 1, NROWS, W2R), lambda t: (0, t, 0, 0)),
            pl.BlockSpec((1, D2), lambda t: (0, 0)),
        ],
        out_specs=pl.BlockSpec((1, N, D2), lambda t: (t, 0, 0)),
        out_shape=jax.ShapeDtypeStruct((SEQ, N, D2), f32),
        compiler_params=_VMEM_BIG,
    )(p2, b2.reshape(1, D2))
    return out


# speedup vs baseline: 1.2366x; 1.2366x over previous
import jax
import jax.numpy as jnp
from jax import lax
from jax.experimental import pallas as pl
from jax.experimental.pallas import tpu as pltpu
from jax.experimental.pallas import tpu_sc as plsc

N = 10000
F = 128
H = 8
O = 8
D1 = H * O
D2 = 128
SEQ = 4
NEG = 0.2
NROWS = 10112
C = 256
NT = 32
NCH = 41
PT = NCH * C
EPAD = NT * PT
RPT = NROWS // 16
W1R = D1 + 16
W2R = D2 + 16
HALF = NROWS // 2
R2 = 5088
QTR = NROWS // 4
QR2 = 2544

f32 = jnp.float32
i32 = jnp.int32

_mesh = plsc.VectorSubcoreMesh(core_axis_name="c", subcore_axis_name="s")
_SC_PARAMS = pltpu.CompilerParams(use_tc_tiling_on_sc=False)
_VMEM_BIG = pltpu.CompilerParams(vmem_limit_bytes=63 * 1024 * 1024)

_GATHER_DN = lax.GatherDimensionNumbers(
    offset_dims=(), collapsed_slice_dims=(0,), start_index_map=(0,))


def _lane_gather(v, idx):
    return lax.gather(v, idx[:, None], _GATHER_DN, slice_sizes=(1,),
                      mode=lax.GatherScatterMode.PROMISE_IN_BOUNDS)


def _make_sc_body(DH, WR, NPH, PHR, DUMMY, DRPT, ZR, ZC, heads):
    def body(idxT, hxf, adpf, out,
             idxb, dscat, adr, hx, zbuf, acc,
             semi, semg, sems):
        cid = lax.axis_index("c")
        sid = lax.axis_index("s")
        wid = sid * 2 + cid
        zv = jnp.zeros((16,), f32)

        @pl.loop(0, ZR)
        def _(r):
            @pl.loop(0, WR, step=16)
            def _(c0):
                zbuf[r, pl.ds(c0, 16)] = zv

        def issue_idx(par, t, g):
            ci = (t * NT + wid) * NCH + g
            pltpu.async_copy(idxT.at[ci], idxb.at[par], semi.at[par])

        def wait_idx(par):
            pltpu.make_async_copy(idxT.at[0], idxb.at[par], semi.at[par]).wait()

        def issue_gathers(par):
            for q in range(2):
                pltpu.async_copy(hxf.at[idxb.at[par, 0, q]],
                                 hx.at[par, pl.ds(q * 128, 128)], semg.at[par])
                pltpu.async_copy(adpf.at[idxb.at[par, 1, q]],
                                 adr.at[par, pl.ds(q * 128, 128)], semg.at[par])

        def wait_gathers(par):
            for q in range(2):
                pltpu.make_async_copy(hxf.at[idxb.at[par, 0, q]],
                                      hx.at[par, pl.ds(q * 128, 128)],
                                      semg.at[par]).wait()
                pltpu.make_async_copy(adpf.at[idxb.at[par, 1, q]],
                                      adr.at[par, pl.ds(q * 128, 128)],
                                      semg.at[par]).wait()

        def issue_scatter(par):
            for q in range(2):
                pltpu.async_copy(hx.at[par, pl.ds(q * 128, 128)],
                                 acc.at[dscat.at[par, q]], sems.at[par],
                                 add=True)

        def wait_scatter(par):
            for q in range(2):
                pltpu.make_async_copy(hx.at[par, pl.ds(q * 128, 128)],
                                      acc.at[dscat.at[par, q]],
                                      sems.at[par]).wait()

        def clamp(par, t, p):
            off = t * NROWS + p * PHR

            for q in range(2):
                @pl.loop(0, 128, step=16)
                def _(k):
                    rel = idxb[par, 1, q, pl.ds(k, 16)] - off
                    inh = (rel >= 0) & (rel < PHR)
                    dscat[par, q, pl.ds(k, 16)] = jnp.where(inh, rel, DUMMY)

        def compute(par):
            pair = lax.iota(i32, 16) >> 3
            lane0 = lax.iota(i32, 16) == 0

            @pl.loop(0, C)
            def _(j):
                gv = hx[par, j, pl.ds(DH, 16)] + adr[par, j]
                e = jnp.exp(jnp.where(gv > 0, gv, gv * NEG))
                if heads:
                    hx[par, j, pl.ds(DH, 16)] = e
                    for r in range(DH // 16):
                        wb = _lane_gather(e, pair + 2 * r)
                        hx[par, j, pl.ds(16 * r, 16)] = (
                            hx[par, j, pl.ds(16 * r, 16)] * wb)
                else:
                    hx[par, j, pl.ds(DH, 16)] = jnp.where(lane0, e, 0.0)
                    for r in range(DH // 16):
                        hx[par, j, pl.ds(16 * r, 16)] = (
                            hx[par, j, pl.ds(16 * r, 16)] * e)

        @pl.loop(0, SEQ)
        def _(t):
            for p in range(NPH):
                for z in range(ZC):
                    pltpu.sync_copy(
                        zbuf, acc.at[pl.ds(sid * (ZR * ZC) + z * ZR, ZR)])
                plsc.subcore_barrier()

                issue_idx(0, t, 0)
                issue_idx(1, t, 1)
                wait_idx(0)
                issue_gathers(0)

                @pl.loop(0, NCH)
                def _(g):
                    par = g & 1
                    wait_gathers(par)
                    clamp(par, t, p)
                    issue_idx(par, t, g + 2)

                    @pl.when(g + 1 < NCH)
                    def _():
                        @pl.when(g >= 1)
                        def _():
                            wait_scatter(1 - par)

                        wait_idx(1 - par)
                        issue_gathers(1 - par)

                    compute(par)
                    issue_scatter(par)

                wait_scatter(0)
                wait_scatter(1)
                wait_idx(0)
                wait_idx(1)
                plsc.subcore_barrier()
                obase = (cid * SEQ + t) * NROWS + p * PHR + sid * DRPT
                pltpu.sync_copy(acc.at[pl.ds(sid * DRPT, DRPT)],
                                out.at[pl.ds(obase, DRPT)])
                plsc.subcore_barrier()

    return body


def _make_sc_kernel(DH, WR, NPH, PHR, DUMMY, DRPT, ACCR, ZR, ZC, heads):
    return pl.kernel(
        _make_sc_body(DH, WR, NPH, PHR, DUMMY, DRPT, ZR, ZC, heads),
        out_type=jax.ShapeDtypeStruct((2 * SEQ * NROWS, WR), f32),
        mesh=_mesh,
        scratch_types=[
            pltpu.VMEM((2, 2, 2, 128), i32), pltpu.VMEM((2, 2, 128), i32),
            pltpu.VMEM((2, C, 16), f32), pltpu.VMEM((2, C, WR), f32),
            pltpu.VMEM((ZR, WR), f32),
            pltpu.VMEM_SHARED((ACCR, WR), f32),
            pltpu.SemaphoreType.DMA((2,)), pltpu.SemaphoreType.DMA((2,)),
            pltpu.SemaphoreType.DMA((2,)),
        ],
        compiler_params=_SC_PARAMS,
    )


def _tc_embed1(x_ref, w_ref, ss_ref, sd_ref, hx_ref, ad_ref):
    h = jnp.dot(x_ref[0], w_ref[...], preferred_element_type=f32)
    a_s = jnp.dot(h, ss_ref[...], preferred_element_type=f32)
    a_d = jnp.dot(h, sd_ref[...], preferred_element_type=f32)
    z8 = jnp.zeros((N, 8), f32)
    hx = jnp.concatenate([h, a_s, z8], 1)
    hx_ref[...] = jnp.concatenate(
        [hx, jnp.zeros((NROWS - N, W1R), f32)], 0)
    ad = jnp.concatenate([a_d, z8], 1)
    ad_ref[...] = jnp.concatenate(
        [ad, jnp.zeros((NROWS - N, 16), f32)], 0)


def _tc_embed2(p_ref, b1_ref, w2_ref, rx_ref, as2_ref, ad2_ref,
               hx_ref, adb_ref):
    m = p_ref[0, 0] + p_ref[1, 0]
    msg = m[:, 0:D1]
    den = m[:, D1:D1 + H]
    den64 = jnp.dot(den, rx_ref[...], preferred_element_type=f32)
    y = msg / (den64 + 1e-16) + b1_ref[...]
    y = jnp.where(y > 0, y, jnp.exp(y) - 1.0)
    h2 = jnp.dot(y, w2_ref[...], preferred_element_type=f32)
    a_s = jnp.sum(h2 * as2_ref[...], axis=-1, keepdims=True)
    a_d = jnp.sum(h2 * ad2_ref[...], axis=-1, keepdims=True)
    hx_ref[...] = jnp.concatenate(
        [h2, jnp.broadcast_to(a_s, (NROWS, 16))], 1)
    adb_ref[...] = jnp.broadcast_to(a_d, (NROWS, 16))


def _tc_final(p_ref, b2_ref, o_ref):
    m = p_ref[0, 0] + p_ref[1, 0]
    v = m[0:N, 0:D2] / (m[0:N, D2:D2 + 1] + 1e-16) + b2_ref[...]
    mx = jnp.max(v, axis=-1, keepdims=True)
    s = v - mx
    o_ref[0] = s - jnp.log(jnp.sum(jnp.exp(s), axis=-1, keepdims=True))


def kernel(x, edge_index, W1, a_src1, a_dst1, b1, W2, a_src2, a_dst2, b2):
    loop_idx = jnp.arange(N, dtype=i32)
    ei = edge_index.astype(i32)
    npad = EPAD - (ei.shape[1] + N)
    padv = jnp.full((npad,), N, i32)
    src = jnp.concatenate([ei[0], loop_idx, padv])
    dst = jnp.concatenate([ei[1], loop_idx, padv])
    toff = (jnp.arange(SEQ, dtype=i32) * NROWS)[:, None]
    srcT = (src[None] + toff).reshape(SEQ, NT, NCH, 2, 128)
    dstT = (dst[None] + toff).reshape(SEQ, NT, NCH, 2, 128)
    idxT = jnp.stack([srcT, dstT], axis=3).reshape(SEQ * NT * NCH, 2, 2, 128)
    idxT = jnp.concatenate([idxT, jnp.full((2, 2, 2, 128), N, i32)], axis=0)

    sc_l1 = _make_sc_kernel(D1, W1R, 2, HALF, HALF, HALF // 16, R2,
                            159, 2, True)
    sc_l2 = _make_sc_kernel(D2, W2R, 4, QTR, QTR, QTR // 16, QR2,
                            159, 1, False)

    eye = jnp.eye(H, dtype=f32)
    s_src = (a_src1[:, :, None] * eye[:, None, :]).reshape(D1, H)
    s_dst = (a_dst1[:, :, None] * eye[:, None, :]).reshape(D1, H)
    rx = jnp.repeat(eye, O, axis=1).reshape(H, D1)

    hx1f, adpf = pl.pallas_call(
        _tc_embed1,
        grid=(SEQ,),
        in_specs=[
            pl.BlockSpec((1, N, F), lambda t: (t, 0, 0)),
            pl.BlockSpec((F, D1), lambda t: (0, 0)),
            pl.BlockSpec((D1, H), lambda t: (0, 0)),
            pl.BlockSpec((D1, H), lambda t: (0, 0)),
        ],
        out_specs=[
            pl.BlockSpec((NROWS, W1R), lambda t: (t, 0)),
            pl.BlockSpec((NROWS, 16), lambda t: (t, 0)),
        ],
        out_shape=[
            jax.ShapeDtypeStruct((SEQ * NROWS, W1R), f32),
            jax.ShapeDtypeStruct((SEQ * NROWS, 16), f32),
        ],
    )(x, W1, s_src, s_dst)

    p1 = sc_l1(idxT, hx1f, adpf)
    p1 = p1.reshape(2, SEQ, NROWS, W1R)

    hx2f, adbf = pl.pallas_call(
        _tc_embed2,
        grid=(SEQ,),
        in_specs=[
            pl.BlockSpec((2, 1, NROWS, W1R), lambda t: (0, t, 0, 0)),
            pl.BlockSpec((1, D1), lambda t: (0, 0)),
            pl.BlockSpec((D1, D2), lambda t: (0, 0)),
            pl.BlockSpec((H, D1), lambda t: (0, 0)),
            pl.BlockSpec((1, D2), lambda t: (0, 0)),
            pl.BlockSpec((1, D2), lambda t: (0, 0)),
        ],
        out_specs=[
            pl.BlockSpec((NROWS, W2R), lambda t: (t, 0)),
            pl.BlockSpec((NROWS, 16), lambda t: (t, 0)),
        ],
        out_shape=[
            jax.ShapeDtypeStruct((SEQ * NROWS, W2R), f32),
            jax.ShapeDtypeStruct((SEQ * NROWS, 16), f32),
        ],
        compiler_params=_VMEM_BIG,
    )(p1, b1.reshape(1, D1), W2, rx, a_src2, a_dst2)

    p2 = sc_l2(idxT, hx2f, adbf)
    p2 = p2.reshape(2, SEQ, NROWS, W2R)

    out = pl.pallas_call(
        _tc_final,
        grid=(SEQ,),
        in_specs=[
            pl.BlockSpec((2, 1, NROWS, W2R), lambda t: (0, t, 0, 0)),
            pl.BlockSpec((1, D2), lambda t: (0, 0)),
        ],
        out_specs=pl.BlockSpec((1, N, D2), lambda t: (t, 0, 0)),
        out_shape=jax.ShapeDtypeStruct((SEQ, N, D2), f32),
        compiler_params=_VMEM_BIG,
    )(p2, b2.reshape(1, D2))
    return out


# speedup vs baseline: 1.4476x; 1.1706x over previous
import jax
import jax.numpy as jnp
from jax import lax
from jax.experimental import pallas as pl
from jax.experimental.pallas import tpu as pltpu
from jax.experimental.pallas import tpu_sc as plsc

N = 10000
F = 128
H = 8
O = 8
D1 = H * O
D2 = 128
SEQ = 4
NEG = 0.2
NROWS = 10112
C = 256
NT = 32
NCH = 41
PT = NCH * C
EPAD = NT * PT
DH = 64
WR = DH + 16
HALF = NROWS // 2
ACCR = 5088
DRPT = HALF // 16
ZR = 159
ZC = 2

f32 = jnp.float32
i32 = jnp.int32

_mesh = plsc.VectorSubcoreMesh(core_axis_name="c", subcore_axis_name="s")
_SC_PARAMS = pltpu.CompilerParams(use_tc_tiling_on_sc=False)
_VMEM_BIG = pltpu.CompilerParams(vmem_limit_bytes=63 * 1024 * 1024)

_GATHER_DN = lax.GatherDimensionNumbers(
    offset_dims=(), collapsed_slice_dims=(0,), start_index_map=(0,))


def _lane_gather(v, idx):
    return lax.gather(v, idx[:, None], _GATHER_DN, slice_sizes=(1,),
                      mode=lax.GatherScatterMode.PROMISE_IN_BOUNDS)


def _make_sc_body(NF, heads):
    def body(idxT, *args):
        tbls = args[0:NF]
        adpf = args[NF]
        out = args[NF + 1]
        (idxb, dscat, adr, hx, zbuf, acc, semi, semg, sems) = args[NF + 2:]
        cid = lax.axis_index("c")
        sid = lax.axis_index("s")
        wid = sid * 2 + cid
        zv = jnp.zeros((16,), f32)

        @pl.loop(0, ZR)
        def _(r):
            @pl.loop(0, WR, step=16)
            def _(c0):
                zbuf[r, pl.ds(c0, 16)] = zv

        def issue_idx(par, t, g):
            ci = (t * NT + wid) * NCH + g
            pltpu.async_copy(idxT.at[ci], idxb.at[par], semi.at[par])

        def wait_idx(par):
            pltpu.make_async_copy(idxT.at[0], idxb.at[par], semi.at[par]).wait()

        def issue_gathers(par, hxf):
            for q in range(2):
                pltpu.async_copy(hxf.at[idxb.at[par, 0, q]],
                                 hx.at[par, pl.ds(q * 128, 128)], semg.at[par])
                pltpu.async_copy(adpf.at[idxb.at[par, 1, q]],
                                 adr.at[par, pl.ds(q * 128, 128)], semg.at[par])

        def wait_gathers(par, hxf):
            for q in range(2):
                pltpu.make_async_copy(hxf.at[idxb.at[par, 0, q]],
                                      hx.at[par, pl.ds(q * 128, 128)],
                                      semg.at[par]).wait()
                pltpu.make_async_copy(adpf.at[idxb.at[par, 1, q]],
                                      adr.at[par, pl.ds(q * 128, 128)],
                                      semg.at[par]).wait()

        def issue_scatter(par):
            for q in range(2):
                pltpu.async_copy(hx.at[par, pl.ds(q * 128, 128)],
                                 acc.at[dscat.at[par, q]], sems.at[par],
                                 add=True)

        def wait_scatter(par):
            for q in range(2):
                pltpu.make_async_copy(hx.at[par, pl.ds(q * 128, 128)],
                                      acc.at[dscat.at[par, q]],
                                      sems.at[par]).wait()

        def clamp(par, t, d):
            off = t * NROWS + d * HALF

            for q in range(2):
                @pl.loop(0, 128, step=16)
                def _(k):
                    rel = idxb[par, 1, q, pl.ds(k, 16)] - off
                    inh = (rel >= 0) & (rel < HALF)
                    dscat[par, q, pl.ds(k, 16)] = jnp.where(inh, rel, HALF)

        def compute(par):
            pair = lax.iota(i32, 16) >> 3
            lane0 = lax.iota(i32, 16) == 0

            @pl.loop(0, C)
            def _(j):
                gv = hx[par, j, pl.ds(DH, 16)] + adr[par, j]
                e = jnp.exp(jnp.where(gv > 0, gv, gv * NEG))
                if heads:
                    hx[par, j, pl.ds(DH, 16)] = e
                    for r in range(DH // 16):
                        wb = _lane_gather(e, pair + 2 * r)
                        hx[par, j, pl.ds(16 * r, 16)] = (
                            hx[par, j, pl.ds(16 * r, 16)] * wb)
                else:
                    hx[par, j, pl.ds(DH, 16)] = jnp.where(lane0, e, 0.0)
                    for r in range(DH // 16):
                        hx[par, j, pl.ds(16 * r, 16)] = (
                            hx[par, j, pl.ds(16 * r, 16)] * e)

        @pl.loop(0, SEQ)
        def _(t):
            for f in range(NF):
                hxf = tbls[f]
                for d in range(2):
                    for z in range(ZC):
                        pltpu.sync_copy(
                            zbuf, acc.at[pl.ds(sid * (ZR * ZC) + z * ZR, ZR)])
                    plsc.subcore_barrier()

                    issue_idx(0, t, 0)
                    issue_idx(1, t, 1)
                    wait_idx(0)
                    issue_gathers(0, hxf)

                    @pl.loop(0, NCH)
                    def _(g):
                        par = g & 1
                        wait_gathers(par, hxf)
                        clamp(par, t, d)
                        issue_idx(par, t, g + 2)

                        @pl.when(g + 1 < NCH)
                        def _():
                            @pl.when(g >= 1)
                            def _():
                                wait_scatter(1 - par)

                            wait_idx(1 - par)
                            issue_gathers(1 - par, hxf)

                        compute(par)
                        issue_scatter(par)

                    wait_scatter(0)
                    wait_scatter(1)
                    wait_idx(0)
                    wait_idx(1)
                    plsc.subcore_barrier()
                    obase = (((cid * SEQ + t) * NF + f) * NROWS
                             + d * HALF + sid * DRPT)
                    pltpu.sync_copy(acc.at[pl.ds(sid * DRPT, DRPT)],
                                    out.at[pl.ds(obase, DRPT)])
                    plsc.subcore_barrier()

    return body


def _make_sc_kernel(NF, heads):
    return pl.kernel(
        _make_sc_body(NF, heads),
        out_type=jax.ShapeDtypeStruct((2 * SEQ * NF * NROWS, WR), f32),
        mesh=_mesh,
        scratch_types=[
            pltpu.VMEM((2, 2, 2, 128), i32), pltpu.VMEM((2, 2, 128), i32),
            pltpu.VMEM((2, C, 16), f32), pltpu.VMEM((2, C, WR), f32),
            pltpu.VMEM((ZR, WR), f32),
            pltpu.VMEM_SHARED((ACCR, WR), f32),
            pltpu.SemaphoreType.DMA((2,)), pltpu.SemaphoreType.DMA((2,)),
            pltpu.SemaphoreType.DMA((2,)),
        ],
        compiler_params=_SC_PARAMS,
    )


def _tc_embed1(x_ref, w_ref, ss_ref, sd_ref, hx_ref, ad_ref):
    h = jnp.dot(x_ref[0], w_ref[...], preferred_element_type=f32)
    a_s = jnp.dot(h, ss_ref[...], preferred_element_type=f32)
    a_d = jnp.dot(h, sd_ref[...], preferred_element_type=f32)
    z8 = jnp.zeros((N, 8), f32)
    hx = jnp.concatenate([h, a_s, z8], 1)
    hx_ref[...] = jnp.concatenate(
        [hx, jnp.zeros((NROWS - N, WR), f32)], 0)
    ad = jnp.concatenate([a_d, z8], 1)
    ad_ref[...] = jnp.concatenate(
        [ad, jnp.zeros((NROWS - N, 16), f32)], 0)


def _tc_embed2(p_ref, b1_ref, w2_ref, rx_ref, as2_ref, ad2_ref,
               hxa_ref, hxb_ref, adb_ref):
    m = p_ref[0, 0] + p_ref[1, 0]
    msg = m[:, 0:D1]
    den = m[:, D1:D1 + H]
    den64 = jnp.dot(den, rx_ref[...], preferred_element_type=f32)
    y = msg / (den64 + 1e-16) + b1_ref[...]
    y = jnp.where(y > 0, y, jnp.exp(y) - 1.0)
    h2 = jnp.dot(y, w2_ref[...], preferred_element_type=f32)
    a_s = jnp.sum(h2 * as2_ref[...], axis=-1, keepdims=True)
    a_d = jnp.sum(h2 * ad2_ref[...], axis=-1, keepdims=True)
    asb = jnp.broadcast_to(a_s, (NROWS, 16))
    hxa_ref[...] = jnp.concatenate([h2[:, 0:DH], asb], 1)
    hxb_ref[...] = jnp.concatenate([h2[:, DH:D2], asb], 1)
    adb_ref[...] = jnp.broadcast_to(a_d, (NROWS, 16))


def _tc_final(p_ref, b2_ref, o_ref):
    m = p_ref[0, 0] + p_ref[1, 0]
    num = jnp.concatenate([m[0, :, 0:DH], m[1, :, 0:DH]], 1)
    v = num / (m[0, :, DH:DH + 1] + 1e-16) + b2_ref[...]
    mx = jnp.max(v, axis=-1, keepdims=True)
    s = v - mx
    o_ref[0] = s - jnp.log(jnp.sum(jnp.exp(s), axis=-1, keepdims=True))


def kernel(x, edge_index, W1, a_src1, a_dst1, b1, W2, a_src2, a_dst2, b2):
    loop_idx = jnp.arange(N, dtype=i32)
    ei = edge_index.astype(i32)
    npad = EPAD - (ei.shape[1] + N)
    padv = jnp.full((npad,), N, i32)
    src = jnp.concatenate([ei[0], loop_idx, padv])
    dst = jnp.concatenate([ei[1], loop_idx, padv])
    toff = (jnp.arange(SEQ, dtype=i32) * NROWS)[:, None]
    srcT = (src[None] + toff).reshape(SEQ, NT, NCH, 2, 128)
    dstT = (dst[None] + toff).reshape(SEQ, NT, NCH, 2, 128)
    idxT = jnp.stack([srcT, dstT], axis=3).reshape(SEQ * NT * NCH, 2, 2, 128)
    idxT = jnp.concatenate([idxT, jnp.full((2, 2, 2, 128), N, i32)], axis=0)

    sc_l1 = _make_sc_kernel(1, True)
    sc_l2 = _make_sc_kernel(2, False)

    eye = jnp.eye(H, dtype=f32)
    s_src = (a_src1[:, :, None] * eye[:, None, :]).reshape(D1, H)
    s_dst = (a_dst1[:, :, None] * eye[:, None, :]).reshape(D1, H)
    rx = jnp.repeat(eye, O, axis=1).reshape(H, D1)

    hx1f, adpf = pl.pallas_call(
        _tc_embed1,
        grid=(SEQ,),
        in_specs=[
            pl.BlockSpec((1, N, F), lambda t: (t, 0, 0)),
            pl.BlockSpec((F, D1), lambda t: (0, 0)),
            pl.BlockSpec((D1, H), lambda t: (0, 0)),
            pl.BlockSpec((D1, H), lambda t: (0, 0)),
        ],
        out_specs=[
            pl.BlockSpec((NROWS, WR), lambda t: (t, 0)),
            pl.BlockSpec((NROWS, 16), lambda t: (t, 0)),
        ],
        out_shape=[
            jax.ShapeDtypeStruct((SEQ * NROWS, WR), f32),
            jax.ShapeDtypeStruct((SEQ * NROWS, 16), f32),
        ],
    )(x, W1, s_src, s_dst)

    p1 = sc_l1(idxT, hx1f, adpf)
    p1 = p1.reshape(2, SEQ, NROWS, WR)

    hx2a, hx2b, adbf = pl.pallas_call(
        _tc_embed2,
        grid=(SEQ,),
        in_specs=[
            pl.BlockSpec((2, 1, NROWS, WR), lambda t: (0, t, 0, 0)),
            pl.BlockSpec((1, D1), lambda t: (0, 0)),
            pl.BlockSpec((D1, D2), lambda t: (0, 0)),
            pl.BlockSpec((H, D1), lambda t: (0, 0)),
            pl.BlockSpec((1, D2), lambda t: (0, 0)),
            pl.BlockSpec((1, D2), lambda t: (0, 0)),
        ],
        out_specs=[
            pl.BlockSpec((NROWS, WR), lambda t: (t, 0)),
            pl.BlockSpec((NROWS, WR), lambda t: (t, 0)),
            pl.BlockSpec((NROWS, 16), lambda t: (t, 0)),
        ],
        out_shape=[
            jax.ShapeDtypeStruct((SEQ * NROWS, WR), f32),
            jax.ShapeDtypeStruct((SEQ * NROWS, WR), f32),
            jax.ShapeDtypeStruct((SEQ * NROWS, 16), f32),
        ],
        compiler_params=_VMEM_BIG,
    )(p1, b1.reshape(1, D1), W2, rx, a_src2, a_dst2)

    p2 = sc_l2(idxT, hx2a, hx2b, adbf)
    p2 = p2.reshape(2, SEQ, 2, NROWS, WR)

    out = pl.pallas_call(
        _tc_final,
        grid=(SEQ, 2),
        in_specs=[
            pl.BlockSpec((2, 1, 2, HALF, WR), lambda t, b: (0, t, 0, b, 0)),
            pl.BlockSpec((1, D2), lambda t, b: (0, 0)),
        ],
        out_specs=pl.BlockSpec((1, HALF, D2), lambda t, b: (t, b, 0)),
        out_shape=jax.ShapeDtypeStruct((SEQ, NROWS, D2), f32),
        compiler_params=_VMEM_BIG,
    )(p2, b2.reshape(1, D2))
    return out[:, 0:N, :]
